# baseline (speedup 1.0000x reference)
import jax
import jax.numpy as jnp
from jax.experimental import pallas as pl
from jax.experimental.pallas import tpu as pltpu

_B, _S, _D = 16, 2048, 512
_NV, _H = 32, 512
_T = _S + _NV
_C = 208
_NT = _T // _C


def _body(seq_ref, emb_ref, w1_ref, b1_ref, w2_ref, b2_ref, inp_ref,
          out_ref, virt_ref):
    b = pl.program_id(0)
    i = pl.program_id(1)

    @pl.when(jnp.logical_and(b == 0, i == 0))
    def _compute_virtual():
        h = jnp.tanh(
            jnp.dot(emb_ref[...], w1_ref[...],
                    preferred_element_type=jnp.float32) + b1_ref[...])
        virt_ref[...] = jnp.dot(
            h, w2_ref[...], preferred_element_type=jnp.float32) + b2_ref[...]

    seq = seq_ref[b]
    t0 = i * _C
    i_w = jax.lax.broadcasted_iota(jnp.int32, (_C, _NV), 0) + t0
    j_w = jax.lax.broadcasted_iota(jnp.int32, (_C, _NV), 1)
    onehot = (i_w - seq == j_w).astype(jnp.float32)
    win_virt = jnp.dot(onehot, virt_ref[...],
                       preferred_element_type=jnp.float32)
    t = jax.lax.broadcasted_iota(jnp.int32, (_C, _D), 0) + t0
    out_ref[0] = jnp.where(t < seq, inp_ref[0], win_virt)


def kernel(inputs, seq_len, embed_table, W1, b1, W2, b2):
    seq_len = seq_len.astype(jnp.int32)

    def _inp_map(b, i, seq_ref):
        return (b, jnp.minimum(i, seq_ref[b] // _C), 0)

    grid_spec = pltpu.PrefetchScalarGridSpec(
        num_scalar_prefetch=1,
        grid=(_B, _NT),
        in_specs=[
            pl.BlockSpec((_NV, _H), lambda b, i, s: (0, 0)),
            pl.BlockSpec((_H, _H), lambda b, i, s: (0, 0)),
            pl.BlockSpec((1, _H), lambda b, i, s: (0, 0)),
            pl.BlockSpec((_H, _D), lambda b, i, s: (0, 0)),
            pl.BlockSpec((1, _D), lambda b, i, s: (0, 0)),
            pl.BlockSpec((1, _C, _D), _inp_map),
        ],
        out_specs=pl.BlockSpec((1, _C, _D), lambda b, i, s: (b, i, 0)),
        scratch_shapes=[pltpu.VMEM((_NV, _D), jnp.float32)],
    )
    out = pl.pallas_call(
        _body,
        grid_spec=grid_spec,
        out_shape=jax.ShapeDtypeStruct((_B, _T, _D), jnp.float32),
    )(seq_len, embed_table, W1, b1.reshape(1, _H), W2, b2.reshape(1, _D),
      inputs)
    return out, seq_len + _NV


# speedup vs baseline: 1.0801x; 1.0801x over previous
import jax
import jax.numpy as jnp
from jax.experimental import pallas as pl
from jax.experimental.pallas import tpu as pltpu

_B, _S, _D = 16, 2048, 512
_NV, _H = 32, 512
_T = _S + _NV
_C = 208
_NT = _T // _C


def _body(seq_ref, emb_ref, w1_ref, b1_ref, w2_ref, b2_ref, inp_ref,
          out_ref, virt_ref):
    b = pl.program_id(0)
    i = pl.program_id(1)

    @pl.when(jnp.logical_and(b == 0, i == 0))
    def _compute_virtual():
        h = jnp.tanh(
            jnp.dot(emb_ref[...], w1_ref[...],
                    preferred_element_type=jnp.float32) + b1_ref[...])
        virt_ref[...] = jnp.dot(
            h, w2_ref[...], preferred_element_type=jnp.float32) + b2_ref[...]

    seq = seq_ref[b]
    t0 = i * _C

    @pl.when(t0 + _C <= seq)
    def _full_copy():
        out_ref[0] = inp_ref[0]

    @pl.when(t0 >= seq + _NV)
    def _zero_fill():
        out_ref[0] = jnp.zeros((_C, _D), jnp.float32)

    @pl.when(jnp.logical_and(t0 + _C > seq, t0 < seq + _NV))
    def _blend():
        i_w = jax.lax.broadcasted_iota(jnp.int32, (_C, _NV), 0) + t0
        j_w = jax.lax.broadcasted_iota(jnp.int32, (_C, _NV), 1)
        onehot = (i_w - seq == j_w).astype(jnp.float32)
        win_virt = jnp.dot(onehot, virt_ref[...],
                           preferred_element_type=jnp.float32)
        t = jax.lax.broadcasted_iota(jnp.int32, (_C, _D), 0) + t0
        out_ref[0] = jnp.where(t < seq, inp_ref[0], win_virt)


def kernel(inputs, seq_len, embed_table, W1, b1, W2, b2):
    seq_len = seq_len.astype(jnp.int32)

    def _inp_map(b, i, seq_ref):
        return (b, jnp.minimum(i, seq_ref[b] // _C), 0)

    grid_spec = pltpu.PrefetchScalarGridSpec(
        num_scalar_prefetch=1,
        grid=(_B, _NT),
        in_specs=[
            pl.BlockSpec((_NV, _H), lambda b, i, s: (0, 0)),
            pl.BlockSpec((_H, _H), lambda b, i, s: (0, 0)),
            pl.BlockSpec((1, _H), lambda b, i, s: (0, 0)),
            pl.BlockSpec((_H, _D), lambda b, i, s: (0, 0)),
            pl.BlockSpec((1, _D), lambda b, i, s: (0, 0)),
            pl.BlockSpec((1, _C, _D), _inp_map),
        ],
        out_specs=pl.BlockSpec((1, _C, _D), lambda b, i, s: (b, i, 0)),
        scratch_shapes=[pltpu.VMEM((_NV, _D), jnp.float32)],
    )
    out = pl.pallas_call(
        _body,
        grid_spec=grid_spec,
        out_shape=jax.ShapeDtypeStruct((_B, _T, _D), jnp.float32),
    )(seq_len, embed_table, W1, b1.reshape(1, _H), W2, b2.reshape(1, _D),
      inputs)
    return out, seq_len + _NV


# speedup vs baseline: 1.4906x; 1.3801x over previous
import jax
import jax.numpy as jnp
from jax.experimental import pallas as pl
from jax.experimental.pallas import tpu as pltpu

_B, _S, _D = 16, 2048, 512
_NV, _H = 32, 512
_T = _S + _NV
_C = 416
_NT = _T // _C


def _body(seq_ref, emb_ref, w1_ref, b1_ref, w2_ref, b2_ref, inp_ref,
          out_ref, virt_ref):
    b = pl.program_id(0)
    i = pl.program_id(1)

    @pl.when(jnp.logical_and(b == 0, i == 0))
    def _compute_virtual():
        h = jnp.tanh(
            jnp.dot(emb_ref[...], w1_ref[...],
                    preferred_element_type=jnp.float32) + b1_ref[...])
        virt_ref[...] = jnp.dot(
            h, w2_ref[...], preferred_element_type=jnp.float32) + b2_ref[...]

    seq = seq_ref[b]
    t0 = i * _C

    @pl.when(t0 + _C <= seq)
    def _full_copy():
        out_ref[0] = inp_ref[0]

    @pl.when(t0 >= seq + _NV)
    def _zero_fill():
        out_ref[0] = jnp.zeros((_C, _D), jnp.float32)

    @pl.when(jnp.logical_and(t0 + _C > seq, t0 < seq + _NV))
    def _blend():
        i_w = jax.lax.broadcasted_iota(jnp.int32, (_C, _NV), 0) + t0
        j_w = jax.lax.broadcasted_iota(jnp.int32, (_C, _NV), 1)
        onehot = (i_w - seq == j_w).astype(jnp.float32)
        win_virt = jnp.dot(onehot, virt_ref[...],
                           preferred_element_type=jnp.float32)
        t = jax.lax.broadcasted_iota(jnp.int32, (_C, _D), 0) + t0
        out_ref[0] = jnp.where(t < seq, inp_ref[0], win_virt)


def kernel(inputs, seq_len, embed_table, W1, b1, W2, b2):
    seq_len = seq_len.astype(jnp.int32)

    def _inp_map(b, i, seq_ref):
        return (b, jnp.minimum(i, seq_ref[b] // _C), 0)

    grid_spec = pltpu.PrefetchScalarGridSpec(
        num_scalar_prefetch=1,
        grid=(_B, _NT),
        in_specs=[
            pl.BlockSpec((_NV, _H), lambda b, i, s: (0, 0)),
            pl.BlockSpec((_H, _H), lambda b, i, s: (0, 0)),
            pl.BlockSpec((1, _H), lambda b, i, s: (0, 0)),
            pl.BlockSpec((_H, _D), lambda b, i, s: (0, 0)),
            pl.BlockSpec((1, _D), lambda b, i, s: (0, 0)),
            pl.BlockSpec((1, _C, _D), _inp_map),
        ],
        out_specs=pl.BlockSpec((1, _C, _D), lambda b, i, s: (b, i, 0)),
        scratch_shapes=[pltpu.VMEM((_NV, _D), jnp.float32)],
    )
    out = pl.pallas_call(
        _body,
        grid_spec=grid_spec,
        out_shape=jax.ShapeDtypeStruct((_B, _T, _D), jnp.float32),
    )(seq_len, embed_table, W1, b1.reshape(1, _H), W2, b2.reshape(1, _D),
      inputs)
    return out, seq_len + _NV


# speedup vs baseline: 1.5161x; 1.0171x over previous
import functools
import jax
import jax.numpy as jnp
from jax import lax
from jax.experimental import pallas as pl
from jax.experimental.pallas import tpu as pltpu
from jax.experimental.pallas import tpu_sc as plsc

_B, _S, _D = 16, 2048, 512
_NV, _H = 32, 512
_T = _S + _NV
_HALF = _T // 2
_CH = 80
_ZCH = 40
_LANES = 16


def _mlp_body(emb_ref, w1_ref, b1_ref, w2_ref, b2_ref, out_ref):
    h = jnp.tanh(
        jnp.dot(emb_ref[...], w1_ref[...],
                preferred_element_type=jnp.float32) + b1_ref[...])
    out_ref[...] = jnp.dot(
        h, w2_ref[...], preferred_element_type=jnp.float32) + b2_ref[...]


def _virtual_rows(embed_table, W1, b1, W2, b2):
    return pl.pallas_call(
        _mlp_body,
        out_shape=jax.ShapeDtypeStruct((_NV, _D), jnp.float32),
    )(embed_table, W1, b1.reshape(1, _H), W2, b2.reshape(1, _D))


def _sc_body(inp_hbm, seq_hbm, virt_hbm, zeros_hbm, out_hbm,
             buf2, vbuf, zbuf, seqv, semA, semB, semZ):
    c = lax.axis_index("c")
    s = lax.axis_index("s")
    wid = s * 2 + c
    b = wid % _B
    half = wid // _B
    row0 = half * _HALF
    r1 = row0 + _HALF

    pltpu.sync_copy(seq_hbm, seqv.at[pl.ds(0, 16)])
    pltpu.sync_copy(virt_hbm, vbuf)
    pltpu.sync_copy(zeros_hbm, zbuf)

    ln = seqv[pl.ds(b, 16)][0]

    copy_rows = jnp.clip(ln - row0, 0, _HALF)
    n_full = copy_rows // _CH

    z0 = jnp.clip(ln + _NV, row0, r1)
    zsu = row0 + ((z0 - row0 + _ZCH - 1) // _ZCH) * _ZCH
    nz = (r1 - zsu) // _ZCH

    def _zdst(j):
        zo = pl.multiple_of(zsu + j * _ZCH, 8)
        return out_hbm.at[b, pl.ds(zo, _ZCH), :]

    def _zfire(j, carry):
        pltpu.make_async_copy(zbuf, _zdst(j), semZ).start()
        return carry
    lax.fori_loop(0, nz, _zfire, 0)

    def _src(k):
        ro = pl.multiple_of(row0 + k * _CH, 8)
        return inp_hbm.at[b, pl.ds(ro, _CH), :]

    def _dst(k):
        ro = pl.multiple_of(row0 + k * _CH, 8)
        return out_hbm.at[b, pl.ds(ro, _CH), :]

    @pl.when(n_full > 0)
    def _prologue0():
        pltpu.make_async_copy(_src(0), buf2.at[0], semA).start()

    @pl.when(n_full > 1)
    def _prologue1():
        pltpu.make_async_copy(_src(1), buf2.at[1], semB).start()

    def _cpair(p_, carry):
        k0 = 2 * p_
        k1 = k0 + 1
        pltpu.make_async_copy(_src(k0), buf2.at[0], semA).wait()
        pltpu.sync_copy(buf2.at[0], _dst(k0))

        @pl.when(k0 + 2 < n_full)
        def _next0():
            pltpu.make_async_copy(_src(k0 + 2), buf2.at[0], semA).start()

        @pl.when(k1 < n_full)
        def _slot1():
            pltpu.make_async_copy(_src(k1), buf2.at[1], semB).wait()
            pltpu.sync_copy(buf2.at[1], _dst(k1))

            @pl.when(k1 + 2 < n_full)
            def _next1():
                pltpu.make_async_copy(_src(k1 + 2), buf2.at[1], semB).start()
        return carry
    lax.fori_loop(0, (n_full + 1) // 2, _cpair, 0)

    owner = jnp.logical_and(ln >= row0, ln < r1)

    @pl.when(owner)
    def _owner_work():
        len8 = (ln // 8) * 8
        pstart = pl.multiple_of(jnp.minimum(len8, _T - _CH), 8)
        p = ln - pstart
        off = row0 + n_full * _CH
        rem8 = pstart - off

        @pl.when(rem8 > 0)
        def _remainder():
            src0 = pl.multiple_of(jnp.minimum(off, _S - _CH), 8)
            delta = off - src0
            pltpu.sync_copy(inp_hbm.at[b, pl.ds(src0, _CH), :], buf2.at[0])
            o = off
            d = delta
            for z in (64, 32, 16, 8):
                take = rem8 & z

                @pl.when(take > 0)
                def _piece(o=o, d=d, z=z):
                    pltpu.sync_copy(
                        buf2.at[0, pl.ds(pl.multiple_of(d, 8), z)],
                        out_hbm.at[b, pl.ds(pl.multiple_of(o, 8), z), :])
                o = o + take
                d = d + take

        pltpu.sync_copy(inp_hbm.at[b, pl.ds(pstart, 48), :],
                        buf2.at[1, pl.ds(0, 48)])

        def _vrow(j, carry):
            for l in range(_D // _LANES):
                buf2[1, p + j, pl.ds(l * _LANES, _LANES)] = (
                    vbuf[j, pl.ds(l * _LANES, _LANES)])
            return carry
        lax.fori_loop(0, _NV, _vrow, 0)

        zero16 = jnp.zeros((_LANES,), jnp.float32)

        def _zrow(j, carry):
            for l in range(_D // _LANES):
                buf2[1, p + _NV + j, pl.ds(l * _LANES, _LANES)] = zero16
            return carry
        lax.fori_loop(0, _CH - _NV - p, _zrow, 0)

        pltpu.sync_copy(buf2.at[1], out_hbm.at[b, pl.ds(pstart, _CH), :])

    def _zdrain(j, carry):
        pltpu.make_async_copy(zbuf, _zdst(j), semZ).wait()
        return carry
    lax.fori_loop(0, nz, _zdrain, 0)


@functools.partial(
    pl.kernel,
    out_type=jax.ShapeDtypeStruct((_B, _T, _D), jnp.float32),
    mesh=plsc.VectorSubcoreMesh(core_axis_name="c", subcore_axis_name="s"),
    scratch_types=[
        pltpu.VMEM((2, _CH, _D), jnp.float32),
        pltpu.VMEM((_NV, _D), jnp.float32),
        pltpu.VMEM((_ZCH, _D), jnp.float32),
        pltpu.VMEM((48,), jnp.int32),
        pltpu.SemaphoreType.DMA,
        pltpu.SemaphoreType.DMA,
        pltpu.SemaphoreType.DMA,
    ],
)
def _sc_assemble(inp_hbm, seq_hbm, virt_hbm, zeros_hbm, out_hbm,
                 buf2, vbuf, zbuf, seqv, semA, semB, semZ):
    _sc_body(inp_hbm, seq_hbm, virt_hbm, zeros_hbm, out_hbm,
             buf2, vbuf, zbuf, seqv, semA, semB, semZ)


def kernel(inputs, seq_len, embed_table, W1, b1, W2, b2):
    seq_len = seq_len.astype(jnp.int32)
    virtual = _virtual_rows(embed_table, W1, b1, W2, b2)
    zeros = jnp.zeros((_ZCH, _D), jnp.float32)
    out = _sc_assemble(inputs, seq_len, virtual, zeros)
    return out, seq_len + _NV
